# Initial kernel scaffold; baseline (speedup 1.0000x reference)
import jax
import jax.numpy as jnp
from jax.experimental import pallas as pl


def kernel(x, W1, b1, W2, b2, W3, b3, Wp1, bp1, Wp2, bp2, fW1, fb1, fW2, fb2, fW3, fb3, edge_index, batch):
    raise NotImplementedError("write your pallas kernel here")


# speedup vs baseline: 108.2245x; 108.2245x over previous
import functools
import math

import jax
import jax.numpy as jnp
from jax import lax
from jax.experimental import pallas as pl
from jax.experimental.pallas import tpu as pltpu

B = 100
NP = 100
EG = 3200
N = B * NP
E = B * EG
FDIM = 128
H = 128
C = 10
K1 = math.ceil(0.5 * NP)
K2 = math.ceil(0.5 * K1)
K3 = math.ceil(0.5 * K2)

G = 2

_NEG_INF = float("-inf")


def _row(v):
    return jnp.transpose(v)


def _sag_body(x_ref, e_ref, W1, b1, W2, b2, W3, b3, Wp1, bp1, Wp2, bp2,
              fW1, fb1, fW2, fb2, fW3, fb3, out_ref):
    i = pl.program_id(0)
    stages = ((W1, b1, Wp1, bp1, K1), (W2, b2, Wp2, bp2, K2),
              (W3, b3, Wp2, bp2, K3))
    for g in range(G):
        gid = i * G + g
        xg = x_ref[0, g]
        src = e_ref[0, g, 0:1, :] - gid * NP
        dst = e_ref[0, g, 1:2, :] - gid * NP
        rows = lax.broadcasted_iota(jnp.int32, (NP, EG), 0)
        ohS = (rows == src).astype(jnp.float32)
        ohD = (rows == dst).astype(jnp.float32)
        cnt = lax.dot_general(ohD, ohS, (((1,), (1,)), ((), ())),
                              preferred_element_type=jnp.float32)

        m = jnp.ones((NP, 1), jnp.float32)
        h = xg
        z = jnp.zeros((1, 3 * H), jnp.float32)
        for (W, b, Wp, bp, k) in stages:
            xw = jnp.dot(h, W[...], preferred_element_type=jnp.float32)
            deg = m * jnp.dot(cnt, m, preferred_element_type=jnp.float32) + m
            dinv = jnp.where(deg > 0, lax.rsqrt(jnp.maximum(deg, 1e-12)), 0.0)
            dm = dinv * m
            A = cnt * dm * _row(dm)
            self_w = dinv * dinv * m
            agg = jnp.dot(A, xw, preferred_element_type=jnp.float32) \
                + self_w * xw
            h = jnp.maximum((agg + b[...]) * m, 0.0)
            xws = jnp.dot(h, Wp[...], preferred_element_type=jnp.float32)
            sc = (jnp.dot(A, xws, preferred_element_type=jnp.float32)
                  + self_w * xws + bp[...]) * m
            scm = jnp.where(m > 0, sc, _NEG_INF)
            scr = _row(scm)
            ii = lax.broadcasted_iota(jnp.int32, (NP, NP), 0)
            jj = lax.broadcasted_iota(jnp.int32, (NP, NP), 1)
            beats = (scr > scm) | ((scr == scm) & (jj < ii))
            rank = jnp.sum(beats.astype(jnp.float32), axis=1, keepdims=True)
            mn = (rank < k).astype(jnp.float32)
            h = h * jnp.tanh(sc) * mn
            ssum = jnp.sum(h, axis=0, keepdims=True)
            mx = jnp.max(jnp.where(mn > 0, h, _NEG_INF), axis=0, keepdims=True)
            z = z + jnp.concatenate([ssum / k, mx, ssum], axis=1)
            m = mn
        z = jnp.maximum(jnp.dot(z, fW1[...], preferred_element_type=jnp.float32)
                        + fb1[...], 0.0)
        z = jnp.maximum(jnp.dot(z, fW2[...], preferred_element_type=jnp.float32)
                        + fb2[...], 0.0)
        z = jnp.dot(z, fW3[...], preferred_element_type=jnp.float32) + fb3[...]
        z = z - jnp.max(z, axis=1, keepdims=True)
        z = z - jnp.log(jnp.sum(jnp.exp(z), axis=1, keepdims=True))
        out_ref[0, g] = z


def _full(shape):
    return pl.BlockSpec(shape, lambda i: (0,) * len(shape))


@jax.jit
def kernel(x, W1, b1, W2, b2, W3, b3, Wp1, bp1, Wp2, bp2,
           fW1, fb1, fW2, fb2, fW3, fb3, edge_index, batch):
    x4 = x.reshape(B // G, G, NP, FDIM)
    e4 = edge_index.astype(jnp.int32).reshape(2, B // G, G, EG)
    e4 = e4.transpose(1, 2, 0, 3)
    args = (x4, e4,
            W1, b1.reshape(1, H), W2, b2.reshape(1, H), W3, b3.reshape(1, H),
            Wp1, bp1.reshape(1, 1), Wp2, bp2.reshape(1, 1),
            fW1, fb1.reshape(1, H), fW2, fb2.reshape(1, H // 2),
            fW3, fb3.reshape(1, C))
    in_specs = [
        pl.BlockSpec((1, G, NP, FDIM), lambda i: (i, 0, 0, 0)),
        pl.BlockSpec((1, G, 2, EG), lambda i: (i, 0, 0, 0)),
    ] + [_full(a.shape) for a in args[2:]]
    out = pl.pallas_call(
        _sag_body,
        grid=(B // G,),
        in_specs=in_specs,
        out_specs=pl.BlockSpec((1, G, 1, C), lambda i: (i, 0, 0, 0)),
        out_shape=jax.ShapeDtypeStruct((B // G, G, 1, C), jnp.float32),
    )(*args)
    return out.reshape(B, C)
